# Initial kernel scaffold; baseline (speedup 1.0000x reference)
import jax
import jax.numpy as jnp
from jax.experimental import pallas as pl


def kernel(input_ids, token_table, pos_table):
    raise NotImplementedError("write your pallas kernel here")


# speedup vs baseline: 3.9705x; 3.9705x over previous
import functools

import jax
import jax.numpy as jnp
from jax import lax
from jax.experimental import pallas as pl
from jax.experimental.pallas import tpu as pltpu
from jax.experimental.pallas import tpu_sc as plsc

VOCAB = 100000
HIDDEN = 128
B, S = 1024, 200
N = B * S
NW = 32
SEQ_PER_W = N // (NW * S)
SPLIT = 104


def _body(ids_hbm, tok_hbm, pos_hbm, out_hbm, idx_v, rows_v, pos_v, sem):
  nc = 2
  wid = lax.axis_index("s") * nc + lax.axis_index("c")

  pltpu.sync_copy(pos_hbm.at[pl.ds(0, S)], pos_v)

  def per_seq(j, _):
    base = (wid * SEQ_PER_W + j) * S
    pltpu.sync_copy(ids_hbm.at[pl.ds(base, S)], idx_v)
    cp0 = pltpu.make_async_copy(
        tok_hbm.at[idx_v.at[pl.ds(0, SPLIT)]], rows_v.at[pl.ds(0, SPLIT)], sem)
    cp1 = pltpu.make_async_copy(
        tok_hbm.at[idx_v.at[pl.ds(SPLIT, S - SPLIT)]],
        rows_v.at[pl.ds(SPLIT, S - SPLIT)], sem)
    cp0.start()
    cp1.start()
    cp0.wait()
    cp1.wait()

    def per_row(r, _):
      for k in range(HIDDEN // 16):
        sl = pl.ds(k * 16, 16)
        plsc.addupdate(rows_v.at[r, sl], pos_v[r, sl])
      return ()

    lax.fori_loop(0, S, per_row, (), unroll=False)

    pltpu.sync_copy(rows_v, out_hbm.at[pl.ds(base, S)])
    return ()

  lax.fori_loop(0, SEQ_PER_W, per_seq, (), unroll=False)


@jax.jit
def kernel(input_ids, token_table, pos_table):
  ids_flat = input_ids.reshape(N)
  mesh = plsc.VectorSubcoreMesh(core_axis_name="c", subcore_axis_name="s")
  run = functools.partial(
      pl.kernel,
      mesh=mesh,
      out_type=jax.ShapeDtypeStruct((N, HIDDEN), jnp.float32),
      scratch_types=[
          pltpu.VMEM((S,), jnp.int32),
          pltpu.VMEM((S, HIDDEN), jnp.float32),
          pltpu.VMEM((S, HIDDEN), jnp.float32),
          pltpu.SemaphoreType.DMA,
      ],
  )(_body)
  out = run(ids_flat, token_table, pos_table)
  return out.reshape(B, S, HIDDEN)


# speedup vs baseline: 6.0553x; 1.5250x over previous
import functools

import jax
import jax.numpy as jnp
from jax import lax
from jax.experimental import pallas as pl
from jax.experimental.pallas import tpu as pltpu
from jax.experimental.pallas import tpu_sc as plsc

VOCAB = 100000
HIDDEN = 128
B, S = 1024, 200
N = B * S
NW = 32
SEQ_PER_W = N // (NW * S)
SPLIT = 104


def _body(ids_hbm, tok_hbm, pos_hbm, out_hbm,
          idx0, idx1, rows0, rows1, pos_v,
          sem_g, sem_s0, sem_s1, sem_i0, sem_i1):
  nc = 2
  wid = lax.axis_index("s") * nc + lax.axis_index("c")
  base0 = wid * (SEQ_PER_W * S)

  pltpu.sync_copy(pos_hbm.at[pl.ds(0, S)], pos_v)

  idx_refs = [idx0, idx1]
  rows_refs = [rows0, rows1]
  sem_s = [sem_s0, sem_s1]
  sem_i = [sem_i0, sem_i1]
  icp = [None, None]
  gcp = [None, None]
  scp = [None, None]

  def idx_load(j, b):
    cp = pltpu.make_async_copy(
        ids_hbm.at[pl.ds(base0 + j * S, S)], idx_refs[b], sem_i[b])
    cp.start()
    return cp

  def gather_start(b):
    cp0 = pltpu.make_async_copy(
        tok_hbm.at[idx_refs[b].at[pl.ds(0, SPLIT)]],
        rows_refs[b].at[pl.ds(0, SPLIT)], sem_g)
    cp1 = pltpu.make_async_copy(
        tok_hbm.at[idx_refs[b].at[pl.ds(SPLIT, S - SPLIT)]],
        rows_refs[b].at[pl.ds(SPLIT, S - SPLIT)], sem_g)
    cp0.start()
    cp1.start()
    return cp0, cp1

  def add_pos(b):
    rows_ref = rows_refs[b]

    def per4(i, _):
      r0 = i * 4
      for rr in range(4):
        for k in range(HIDDEN // 16):
          sl = pl.ds(k * 16, 16)
          plsc.addupdate(rows_ref.at[r0 + rr, sl], pos_v[r0 + rr, sl])
      return ()

    lax.fori_loop(0, S // 4, per4, (), unroll=False)

  def scatter_start(j, b):
    cp = pltpu.make_async_copy(
        rows_refs[b], out_hbm.at[pl.ds(base0 + j * S, S)], sem_s[b])
    cp.start()
    return cp

  icp[0] = idx_load(0, 0)
  for j in range(SEQ_PER_W):
    b = j % 2
    icp[b].wait()
    if scp[b] is not None:
      scp[b].wait()
    gcp[b] = gather_start(b)
    if j == 0:
      icp[1] = idx_load(1, 1)
    else:
      gcp[1 - b][0].wait()
      gcp[1 - b][1].wait()
      if j + 1 < SEQ_PER_W:
        icp[1 - b] = idx_load(j + 1, 1 - b)
      add_pos(1 - b)
      scp[1 - b] = scatter_start(j - 1, 1 - b)

  last = SEQ_PER_W - 1
  b = last % 2
  gcp[b][0].wait()
  gcp[b][1].wait()
  add_pos(b)
  scp[b] = scatter_start(last, b)
  scp[0].wait()
  scp[1].wait()


@jax.jit
def kernel(input_ids, token_table, pos_table):
  ids_flat = input_ids.reshape(N)
  mesh = plsc.VectorSubcoreMesh(core_axis_name="c", subcore_axis_name="s")
  run = functools.partial(
      pl.kernel,
      mesh=mesh,
      out_type=jax.ShapeDtypeStruct((N, HIDDEN), jnp.float32),
      scratch_types=[
          pltpu.VMEM((S,), jnp.int32),
          pltpu.VMEM((S,), jnp.int32),
          pltpu.VMEM((S, HIDDEN), jnp.float32),
          pltpu.VMEM((S, HIDDEN), jnp.float32),
          pltpu.VMEM((S, HIDDEN), jnp.float32),
          pltpu.SemaphoreType.DMA,
          pltpu.SemaphoreType.DMA,
          pltpu.SemaphoreType.DMA,
          pltpu.SemaphoreType.DMA,
          pltpu.SemaphoreType.DMA,
      ],
  )(_body)
  out = run(ids_flat, token_table, pos_table)
  return out.reshape(B, S, HIDDEN)
